# baseline (speedup 1.0000x reference)
import functools

import numpy as np

import jax
import jax.numpy as jnp
from jax import lax
from jax.experimental import pallas as pl
from jax.experimental.pallas import tpu as pltpu
from jax.experimental.pallas import tpu_sc as plsc

N = 10000
E = 320000
D = 128

NC = 2
NS = 16
NW = NC * NS
EPW = E // NW
CHUNK = 80
NCHUNK = EPW // CHUNK
NPAD = 10240
RPS = NPAD // NS
NSS = 4
NDS = 6

_ev = np.arange(0, 32, 2)
_PERM = np.concatenate([np.concatenate([j * 32 + _ev, j * 32 + _ev + 1])
                        for j in range(D // 32)])
_INV_COLS = np.argsort(_PERM)


def _sc_segment_sum(hb, src, dst):
  mesh = plsc.VectorSubcoreMesh(core_axis_name="c", subcore_axis_name="s")

  @functools.partial(
      pl.kernel,
      out_type=jax.ShapeDtypeStruct((NC, NPAD, D), jnp.float32),
      mesh=mesh,
      compiler_params=pltpu.CompilerParams(use_tc_tiling_on_sc=False),
      scratch_types=(
          [pltpu.VMEM_SHARED((NPAD, D), jnp.float32)]
          + [pltpu.VMEM((CHUNK, D), jnp.float32)] * 2
          + [pltpu.VMEM((CHUNK, D // 2), jnp.int32)] * 2
          + [pltpu.VMEM((CHUNK,), jnp.int32)] * NSS
          + [pltpu.VMEM((CHUNK,), jnp.int32)] * NDS
          + [pltpu.SemaphoreType.DMA] * (4 + NSS + NDS)
      ),
  )
  def seg_sum(h_hbm, src_hbm, dst_hbm, out_hbm, acc, *scr):
    stage = scr[:2]
    bfb = scr[2:4]
    sslot = scr[4:4 + NSS]
    dslot = scr[4 + NSS:4 + NSS + NDS]
    sems = scr[4 + NSS + NDS:]
    gsem = sems[:2]
    ssem = sems[2:4]
    isem = sems[4:4 + NSS]
    dsem = sems[4 + NSS:]

    c = lax.axis_index("c")
    s = lax.axis_index("s")
    wid = c * NS + s
    ebase = wid * EPW
    rbase = s * RPS

    def fetch_src(i, q):
      pltpu.async_copy(src_hbm.at[pl.ds(ebase + i * CHUNK, CHUNK)],
                       sslot[q], isem[q])

    def fetch_dst(i, q):
      pltpu.async_copy(dst_hbm.at[pl.ds(ebase + i * CHUNK, CHUNK)],
                       dslot[q], dsem[q])

    def wait_idx(slot, sem):
      pltpu.make_async_copy(src_hbm.at[pl.ds(ebase, CHUNK)], slot, sem).wait()

    def gather(q, b):
      pltpu.async_copy(h_hbm.at[sslot[q]], bfb[b], gsem[b])

    def wait_gather(b):
      pltpu.make_async_copy(h_hbm.at[sslot[0]], bfb[b], gsem[b]).wait()

    def wait_scatter(b):
      pltpu.make_async_copy(out_hbm.at[0, pl.ds(0, CHUNK)], stage[b],
                            ssem[b]).wait()

    def convert(b):
      def conv4(rr, carry):
        for t in range(4):
          r = rr * 4 + t
          for j in range(D // 32):
            u = bfb[b][r, pl.ds(j * 16, 16)]
            lo = lax.bitcast_convert_type(u << jnp.int32(16), jnp.float32)
            hi = lax.bitcast_convert_type(u & jnp.int32(-65536), jnp.float32)
            stage[b][r, pl.ds(j * 32, 16)] = lo
            stage[b][r, pl.ds(j * 32 + 16, 16)] = hi
        return carry

      lax.fori_loop(0, CHUNK // 4, conv4, 0)

    def step(i, im, fs_on=True, sw_on=True, c_on=True, d_on=True):
      b, q4, q6 = im % 2, im % NSS, im % NDS
      wait_gather(b)
      if fs_on:
        fetch_src(i + 4, q4)
      if sw_on:
        wait_scatter(b)
      convert(b)
      wait_idx(dslot[q6], dsem[q6])
      pltpu.async_copy(stage[b], acc.at[dslot[q6]], ssem[b], add=True)
      if c_on:
        fetch_dst(i + 5, (q6 + 5) % NDS)
      if d_on:
        wait_idx(sslot[(q4 + 2) % NSS], isem[(q4 + 2) % NSS])
        gather((q4 + 2) % NSS, b)

    zv = jnp.zeros((16,), jnp.float32)

    def zrow(r, carry):
      for j in range(D // 16):
        stage[0][r, pl.ds(j * 16, 16)] = zv
      return carry

    lax.fori_loop(0, CHUNK, zrow, 0)
    for r2 in range(RPS // CHUNK):
      pltpu.sync_copy(stage[0], acc.at[pl.ds(rbase + r2 * CHUNK, CHUNK)])
    for q in range(NSS):
      fetch_src(q, q)
    for q in range(NDS - 1):
      fetch_dst(q, q)
    plsc.subcore_barrier()
    for b in range(2):
      wait_idx(sslot[b], isem[b])
      gather(b, b)

    step(0, 0, sw_on=False)
    step(1, 1, sw_on=False)

    def outer(io, carry):
      for k in range(12):
        step(12 * io + 2 + k, 2 + k)
      return carry

    lax.fori_loop(0, 9, outer, 0)
    for i in range(110, 125):
      step(i, i % 12,
           fs_on=(i <= 120),
           c_on=(i <= 119),
           d_on=(i <= 122))
    wait_scatter(1)
    wait_scatter(0)

    plsc.subcore_barrier()
    pltpu.sync_copy(acc.at[pl.ds(rbase, RPS)],
                    out_hbm.at[c, pl.ds(rbase, RPS)])

  return seg_sum(hb, src, dst)


def _tc_combine(x, maskf, W, b, partials):
  BLK = 2000

  def body(x_ref, m_ref, w_ref, b_ref, p0_ref, p1_ref, o_ref):
    hin = (lax.dot_general(x_ref[...], w_ref[...], (((1,), (1,)), ((), ())),
                           preferred_element_type=jnp.float32)
           + b_ref[...]) * m_ref[...]
    o_ref[...] = jnp.tanh(hin + p0_ref[0] + p1_ref[0])

  return pl.pallas_call(
      body,
      grid=(N // BLK,),
      in_specs=[
          pl.BlockSpec((BLK, D), lambda i: (i, 0)),
          pl.BlockSpec((BLK, 1), lambda i: (i, 0)),
          pl.BlockSpec((D, D), lambda i: (0, 0)),
          pl.BlockSpec((1, D), lambda i: (0, 0)),
          pl.BlockSpec((1, BLK, D), lambda i: (0, i, 0)),
          pl.BlockSpec((1, BLK, D), lambda i: (1, i, 0)),
      ],
      out_specs=pl.BlockSpec((BLK, D), lambda i: (i, 0)),
      out_shape=jax.ShapeDtypeStruct((N, D), jnp.float32),
  )(x, maskf, W, b, partials, partials)


def kernel(x, h, mask, edge_index, W_in, b_in):
  src = edge_index[0].astype(jnp.int32)
  dst = edge_index[1].astype(jnp.int32)
  hb = h.astype(jnp.bfloat16)[:, _INV_COLS]
  hb32 = lax.bitcast_convert_type(hb.reshape(N, D // 2, 2), jnp.int32)
  partials = _sc_segment_sum(hb32, src, dst)
  maskf = mask.astype(jnp.float32)[:, None]
  return _tc_combine(x, maskf, W_in, b_in.reshape(1, D), partials)


# speedup vs baseline: 2.0507x; 2.0507x over previous
import functools

import jax
import jax.numpy as jnp
from jax import lax
from jax.experimental import pallas as pl
from jax.experimental.pallas import tpu as pltpu
from jax.experimental.pallas import tpu_sc as plsc

N = 10000
E = 320000
D = 128

NC = 2
NS = 16
NW = NC * NS
EPW = E // NW
CHUNK = 80
NCHUNK = EPW // CHUNK
NPAD = 10240
RPS = NPAD // NS
NROW = 3
NSS = 4
NDS = 6


def _sc_segment_sum(h, src, dst):
  mesh = plsc.VectorSubcoreMesh(core_axis_name="c", subcore_axis_name="s")

  @functools.partial(
      pl.kernel,
      out_type=jax.ShapeDtypeStruct((NC, NPAD, D), jnp.float32),
      mesh=mesh,
      scratch_types=(
          [pltpu.VMEM_SHARED((NPAD, D), jnp.float32)]
          + [pltpu.VMEM((CHUNK, D), jnp.float32)] * NROW
          + [pltpu.VMEM((CHUNK,), jnp.int32)] * NSS
          + [pltpu.VMEM((CHUNK,), jnp.int32)] * NDS
          + [pltpu.SemaphoreType.DMA] * (2 * NROW + NSS + NDS)
      ),
  )
  def seg_sum(h_hbm, src_hbm, dst_hbm, out_hbm, acc, *scr):
    rows = scr[:NROW]
    sslot = scr[NROW:NROW + NSS]
    dslot = scr[NROW + NSS:NROW + NSS + NDS]
    sems = scr[NROW + NSS + NDS:]
    gsem = sems[:NROW]
    ssem = sems[NROW:2 * NROW]
    isem = sems[2 * NROW:2 * NROW + NSS]
    dsem = sems[2 * NROW + NSS:]

    c = lax.axis_index("c")
    s = lax.axis_index("s")
    wid = c * NS + s
    ebase = wid * EPW
    rbase = s * RPS

    def fetch_src(i, q):
      pltpu.async_copy(src_hbm.at[pl.ds(ebase + i * CHUNK, CHUNK)],
                       sslot[q], isem[q])

    def fetch_dst(i, q):
      pltpu.async_copy(dst_hbm.at[pl.ds(ebase + i * CHUNK, CHUNK)],
                       dslot[q], dsem[q])

    def wait_idx(slot, sem):
      pltpu.make_async_copy(src_hbm.at[pl.ds(ebase, CHUNK)], slot, sem).wait()

    def gather(q, b):
      pltpu.async_copy(h_hbm.at[sslot[q]], rows[b], gsem[b])

    def wait_sem(b, sem_ring):
      pltpu.make_async_copy(h_hbm.at[sslot[0]], rows[b], sem_ring[b]).wait()

    def step(i, im, a_on=True, fs_on=True, b_on=True, c_on=True, d_on=True):
      b, q4, q6 = im % NROW, im % NSS, im % NDS
      if a_on:
        wait_sem(b, gsem)
        if fs_on:
          fetch_src(i + 4, q4)
        wait_idx(dslot[q6], dsem[q6])
        pltpu.async_copy(rows[b], acc.at[dslot[q6]], ssem[b], add=True)
      if b_on:
        wait_sem((im + 2) % NROW, ssem)
      if c_on:
        fetch_dst(i + 5, (im + 5) % NDS)
      if d_on:
        wait_idx(sslot[(q4 + 2) % NSS], isem[(q4 + 2) % NSS])
        gather((q4 + 2) % NSS, (im + 2) % NROW)

    zv = jnp.zeros((16,), jnp.float32)

    def zrow(r, carry):
      for j in range(D // 16):
        rows[0][r, pl.ds(j * 16, 16)] = zv
      return carry

    lax.fori_loop(0, CHUNK, zrow, 0)
    for r2 in range(RPS // CHUNK):
      pltpu.sync_copy(rows[0], acc.at[pl.ds(rbase + r2 * CHUNK, CHUNK)])
    for q in range(NSS):
      fetch_src(q, q)
    for q in range(NDS - 1):
      fetch_dst(q, q)
    plsc.subcore_barrier()
    for b in range(2):
      wait_idx(sslot[b], isem[b])
      gather(b, b)

    step(0, 0, b_on=False)
    step(1, 1, b_on=(NROW == 3))

    def outer(io, carry):
      for k in range(12):
        step(12 * io + 2 + k, 2 + k)
      return carry

    lax.fori_loop(0, 9, outer, 0)
    for i in range(110, 125 + NROW - 2):
      step(i, i % 12,
           a_on=(i <= 124),
           fs_on=(i <= 120),
           b_on=(i - (NROW - 2) <= 124),
           c_on=(i <= 119),
           d_on=(i <= 122))

    plsc.subcore_barrier()
    pltpu.sync_copy(acc.at[pl.ds(rbase, RPS)],
                    out_hbm.at[c, pl.ds(rbase, RPS)])

  return seg_sum(h, src, dst)


def _tc_combine(x, maskf, W, b, partials):
  BLK = 2000

  def body(x_ref, m_ref, w_ref, b_ref, p0_ref, p1_ref, o_ref):
    hin = (lax.dot_general(x_ref[...], w_ref[...], (((1,), (1,)), ((), ())),
                           preferred_element_type=jnp.float32)
           + b_ref[...]) * m_ref[...]
    o_ref[...] = jnp.tanh(hin + p0_ref[0] + p1_ref[0])

  return pl.pallas_call(
      body,
      grid=(N // BLK,),
      in_specs=[
          pl.BlockSpec((BLK, D), lambda i: (i, 0)),
          pl.BlockSpec((BLK, 1), lambda i: (i, 0)),
          pl.BlockSpec((D, D), lambda i: (0, 0)),
          pl.BlockSpec((1, D), lambda i: (0, 0)),
          pl.BlockSpec((1, BLK, D), lambda i: (0, i, 0)),
          pl.BlockSpec((1, BLK, D), lambda i: (1, i, 0)),
      ],
      out_specs=pl.BlockSpec((BLK, D), lambda i: (i, 0)),
      out_shape=jax.ShapeDtypeStruct((N, D), jnp.float32),
  )(x, maskf, W, b, partials, partials)


def kernel(x, h, mask, edge_index, W_in, b_in):
  src = edge_index[0].astype(jnp.int32)
  dst = edge_index[1].astype(jnp.int32)
  partials = _sc_segment_sum(h, src, dst)
  maskf = mask.astype(jnp.float32)[:, None]
  return _tc_combine(x, maskf, W_in, b_in.reshape(1, D), partials)
